# baseline (speedup 1.0000x reference)
import functools

import jax
import jax.numpy as jnp
from jax import lax
from jax.experimental import pallas as pl
from jax.experimental.pallas import tpu as pltpu
from jax.experimental.pallas import tpu_sc as plsc

HIDDEN = 128
MAX_POS = 4096
BATCH = 4
SEQ = 4096

NC, NS, L = 2, 16, 16
NW = NC * NS
SRANGE = SEQ // NW
PIECE = 32
NGROUP = SRANGE // PIECE
CL = HIDDEN // L
RING = 2 * BATCH


def _sc_embed(input_ids, token_table, position_table):
    mesh = plsc.VectorSubcoreMesh(core_axis_name="c", subcore_axis_name="s")

    @functools.partial(
        pl.kernel,
        mesh=mesh,
        out_type=jax.ShapeDtypeStruct((BATCH, SEQ, HIDDEN), jnp.float32),
        scratch_types=[
            pltpu.VMEM((BATCH, SRANGE), jnp.int32),
            pltpu.VMEM((SRANGE, HIDDEN), jnp.float32),
            pltpu.VMEM((BATCH * SRANGE, HIDDEN), jnp.float32),
            pltpu.SemaphoreType.DMA,
            pltpu.SemaphoreType.DMA,
            pltpu.SemaphoreType.DMA((RING,)),
            pltpu.SemaphoreType.DMA((RING,)),
        ],
    )
    def body(ids_hbm, tok_hbm, pos_hbm, out_hbm, idx_v, pos_v, tok_v,
             isem, psem, gsem, ssem):
        wid = lax.axis_index("s") * NC + lax.axis_index("c")
        ss = wid * SRANGE

        def gather(b, h):
            pltpu.async_copy(
                tok_hbm.at[idx_v.at[b, pl.ds(h * PIECE, PIECE)]],
                tok_v.at[pl.ds(b * SRANGE + h * PIECE, PIECE)],
                gsem.at[(h % 2) * BATCH + b])

        def gather_wait(b, h):
            pltpu.make_async_copy(
                tok_hbm.at[pl.ds(0, PIECE)],
                tok_v.at[pl.ds(b * SRANGE + h * PIECE, PIECE)],
                gsem.at[(h % 2) * BATCH + b]).wait()

        def store(b, h):
            pltpu.async_copy(
                tok_v.at[pl.ds(b * SRANGE + h * PIECE, PIECE)],
                out_hbm.at[b, pl.ds(ss + h * PIECE, PIECE)],
                ssem.at[(h % 2) * BATCH + b])

        def store_wait(b, h):
            pltpu.make_async_copy(
                tok_hbm.at[pl.ds(0, PIECE)],
                out_hbm.at[b, pl.ds(ss + h * PIECE, PIECE)],
                ssem.at[(h % 2) * BATCH + b]).wait()

        icps = [
            pltpu.async_copy(ids_hbm.at[b, pl.ds(ss, SRANGE)],
                             idx_v.at[b], isem)
            for b in range(BATCH)
        ]
        pcp = pltpu.async_copy(pos_hbm.at[pl.ds(ss, SRANGE)], pos_v, psem)
        for cp in icps:
            cp.wait()

        for h in range(2):
            for b in range(BATCH):
                gather(b, h)

        for h in range(NGROUP):
            for b in range(BATCH):
                gather_wait(b, h)
            if h + 2 < NGROUP:
                for b in range(BATCH):
                    gather(b, h + 2)
            if h == 0:
                pcp.wait()
            ph = h * PIECE

            @plsc.parallel_loop(0, PIECE, unroll=2)
            def add_srow(r, ph=ph):
                sr = ph + r
                pv = [pos_v[sr, pl.ds(c * L, L)] for c in range(CL)]
                for b in range(BATCH):
                    row = b * SRANGE + sr
                    for c in range(CL):
                        cs = pl.ds(c * L, L)
                        tok_v[row, cs] = tok_v[row, cs] + pv[c]

            for b in range(BATCH):
                if h >= 2:
                    store_wait(b, h - 2)
                store(b, h)

        for h in range(NGROUP - 2, NGROUP):
            for b in range(BATCH):
                store_wait(b, h)

    return body(input_ids, token_table, position_table)


def kernel(input_ids, token_table, position_table):
    return _sc_embed(input_ids.astype(jnp.int32), token_table,
                     position_table)


# speedup vs baseline: 1.0049x; 1.0049x over previous
import functools

import jax
import jax.numpy as jnp
from jax import lax
from jax.experimental import pallas as pl
from jax.experimental.pallas import tpu as pltpu
from jax.experimental.pallas import tpu_sc as plsc

HIDDEN = 128
MAX_POS = 4096
BATCH = 4
SEQ = 4096

NC, NS, L = 2, 16, 16
NW = NC * NS
SRANGE = SEQ // NW
PIECE = 64
NSPLIT = SRANGE // PIECE
NPIECE = BATCH * NSPLIT
ROWS = NPIECE * PIECE


def _sc_embed(input_ids, token_table, position_table):
    mesh = plsc.VectorSubcoreMesh(core_axis_name="c", subcore_axis_name="s")

    @functools.partial(
        pl.kernel,
        mesh=mesh,
        out_type=jax.ShapeDtypeStruct((BATCH, SEQ, HIDDEN), jnp.float32),
        scratch_types=[
            pltpu.VMEM((BATCH, SRANGE), jnp.int32),
            pltpu.VMEM((SRANGE, HIDDEN), jnp.float32),
            pltpu.VMEM((ROWS, HIDDEN), jnp.float32),
            pltpu.SemaphoreType.DMA,
            pltpu.SemaphoreType.DMA,
            pltpu.SemaphoreType.DMA((NPIECE,)),
            pltpu.SemaphoreType.DMA((NPIECE,)),
        ],
    )
    def body(ids_hbm, tok_hbm, pos_hbm, out_hbm, idx_v, pos_v, tok_v,
             isem, psem, gsem, ssem):
        wid = lax.axis_index("s") * NC + lax.axis_index("c")
        ss = wid * SRANGE

        icp = pltpu.async_copy(ids_hbm.at[wid], idx_v, isem)
        pcp = pltpu.async_copy(pos_hbm.at[pl.ds(ss, SRANGE)], pos_v, psem)
        icp.wait()

        def issue(k, carry):
            b, h = k // NSPLIT, lax.rem(k, NSPLIT)
            pltpu.async_copy(
                tok_hbm.at[idx_v.at[b, pl.ds(h * PIECE, PIECE)]],
                tok_v.at[pl.ds(k * PIECE, PIECE)], gsem.at[k])
            return carry

        lax.fori_loop(0, NPIECE, issue, 0)
        pcp.wait()

        def process(k, carry):
            b, h = k // NSPLIT, lax.rem(k, NSPLIT)
            dst = tok_v.at[pl.ds(k * PIECE, PIECE)]
            pltpu.make_async_copy(tok_hbm.at[pl.ds(0, PIECE)], dst,
                                  gsem.at[k]).wait()
            row0 = k * PIECE
            ph = h * PIECE

            @plsc.parallel_loop(0, PIECE, unroll=8)
            def add_row(r):
                row = row0 + r
                prow = ph + r
                for c in range(HIDDEN // L):
                    cs = pl.ds(c * L, L)
                    tok_v[row, cs] = tok_v[row, cs] + pos_v[prow, cs]
            pltpu.async_copy(
                dst, out_hbm.at[b, pl.ds(ss + h * PIECE, PIECE)],
                ssem.at[k])
            return carry

        lax.fori_loop(0, NPIECE, process, 0)

        def drain(k, carry):
            pltpu.make_async_copy(
                tok_hbm.at[pl.ds(0, PIECE)],
                out_hbm.at[0, pl.ds(0, PIECE)], ssem.at[k]).wait()
            return carry

        lax.fori_loop(0, NPIECE, drain, 0)

    return body(input_ids, token_table, position_table)


def kernel(input_ids, token_table, position_table):
    ids_r = jnp.transpose(
        input_ids.astype(jnp.int32).reshape(BATCH, NW, SRANGE), (1, 0, 2))
    return _sc_embed(ids_r, token_table, position_table)


# speedup vs baseline: 1.0159x; 1.0109x over previous
import functools

import jax
import jax.numpy as jnp
from jax import lax
from jax.experimental import pallas as pl
from jax.experimental.pallas import tpu as pltpu
from jax.experimental.pallas import tpu_sc as plsc

HIDDEN = 128
MAX_POS = 4096
BATCH = 4
SEQ = 4096

NC, NS, L = 2, 16, 16
NW = NC * NS
SRANGE = SEQ // NW
PIECE = 64
NSPLIT = SRANGE // PIECE
NPIECE = BATCH * NSPLIT
ROWS = NPIECE * PIECE


def _sc_embed(input_ids, token_table, position_table):
    mesh = plsc.VectorSubcoreMesh(core_axis_name="c", subcore_axis_name="s")

    @functools.partial(
        pl.kernel,
        mesh=mesh,
        out_type=jax.ShapeDtypeStruct((BATCH, SEQ, HIDDEN), jnp.float32),
        scratch_types=[
            pltpu.VMEM((BATCH, SRANGE), jnp.int32),
            pltpu.VMEM((SRANGE, HIDDEN), jnp.float32),
            pltpu.VMEM((ROWS, HIDDEN), jnp.float32),
            pltpu.SemaphoreType.DMA,
            pltpu.SemaphoreType.DMA,
            pltpu.SemaphoreType.DMA((NPIECE,)),
            pltpu.SemaphoreType.DMA((NPIECE,)),
        ],
    )
    def body(ids_hbm, tok_hbm, pos_hbm, out_hbm, idx_v, pos_v, tok_v,
             isem, psem, gsem, ssem):
        wid = lax.axis_index("s") * NC + lax.axis_index("c")
        ss = wid * SRANGE

        icp = pltpu.async_copy(ids_hbm.at[wid], idx_v, isem)
        pcp = pltpu.async_copy(pos_hbm.at[pl.ds(ss, SRANGE)], pos_v, psem)
        icp.wait()

        def issue(k, carry):
            b, h = k // NSPLIT, lax.rem(k, NSPLIT)
            pltpu.async_copy(
                tok_hbm.at[idx_v.at[b, pl.ds(h * PIECE, PIECE)]],
                tok_v.at[pl.ds(k * PIECE, PIECE)], gsem.at[k])
            return carry

        lax.fori_loop(0, NPIECE, issue, 0)
        pcp.wait()

        def process(k, carry):
            b, h = k // NSPLIT, lax.rem(k, NSPLIT)
            dst = tok_v.at[pl.ds(k * PIECE, PIECE)]
            pltpu.make_async_copy(tok_hbm.at[pl.ds(0, PIECE)], dst,
                                  gsem.at[k]).wait()
            row0 = k * PIECE
            ph = h * PIECE

            @plsc.parallel_loop(0, PIECE, unroll=4)
            def add_row(r):
                row = row0 + r
                prow = ph + r
                for c in range(HIDDEN // L):
                    cs = pl.ds(c * L, L)
                    tok_v[row, cs] = tok_v[row, cs] + pos_v[prow, cs]
            pltpu.async_copy(
                dst, out_hbm.at[b, pl.ds(ss + h * PIECE, PIECE)],
                ssem.at[k])
            return carry

        lax.fori_loop(0, NPIECE, process, 0)

        def drain(k, carry):
            pltpu.make_async_copy(
                tok_hbm.at[pl.ds(0, PIECE)],
                out_hbm.at[0, pl.ds(0, PIECE)], ssem.at[k]).wait()
            return carry

        lax.fori_loop(0, NPIECE, drain, 0)

    return body(input_ids, token_table, position_table)


def kernel(input_ids, token_table, position_table):
    ids_r = jnp.transpose(
        input_ids.astype(jnp.int32).reshape(BATCH, NW, SRANGE), (1, 0, 2))
    return _sc_embed(ids_r, token_table, position_table)
